# baseline (speedup 1.0000x reference)
import functools

import jax
import jax.numpy as jnp
from jax import lax
from jax.experimental import pallas as pl
from jax.experimental.pallas import tpu as pltpu
from jax.experimental.pallas import tpu_sc as plsc

_N_GRAPHS = 8
N = 50000
E = 800000
NPAD = 50048
EPAD = 819200
EROWS = EPAD // 128
NS = 16
ROWS_PER_SUB = EROWS // NS
CH_ROWS = 8
N_CHUNKS = ROWS_PER_SUB // CH_ROWS
NODE_SLICE = NPAD // NS
WH = 16


def _sweep_body(layer1, two_pass, sp_names, *args):
    (src2d, dst2d, c2d, asrc_h, adst_h, winv_h, hs_h,
     ea0_h, ea1_h, ea2_h, zr_h, zv_h, ones_h,
     e2d, den_h, out_h, aux0_h, aux1_h, aux2_h, aux3_h) = args[:20]
    vmem = args[20:46]
    out_sp = args[46]
    sp = dict(zip(sp_names, args[47:47 + len(sp_names)]))
    gsem0, gsem1, nsem0, nsem1, ssem = args[47 + len(sp_names):]
    asrc_sp = sp.get("asrc")
    adst_sp = sp.get("adst")
    winv_sp = sp.get("winv")
    den_sp = sp.get("den")
    cnt_sp = sp.get("cnt")
    s0_sp = sp.get("s0")
    s1_sp = sp.get("s1")
    s2_sp = sp.get("s2")
    c = lax.axis_index("c")
    s = lax.axis_index("s")
    bufs = [vmem[0:12], vmem[12:24]]
    zv, onesv = vmem[24:26]
    gsems = [gsem0, gsem1]
    nsems = [nsem0, nsem1]
    rows = bufs[0][8]

    nbase = s * NODE_SLICE
    if True:
        pairs = [(asrc_h, asrc_sp), (adst_h, adst_sp)]
        if not layer1:
            pairs.append((winv_h, winv_sp))
        for src_h, dst_sp in pairs:
            for k in range(4):
                r0 = nbase + k * 1024
                nr = 1024 if k < 3 else NODE_SLICE - 3 * 1024
                pltpu.sync_copy(src_h.at[pl.ds(r0, nr)], zv.at[pl.ds(0, nr)])
                pltpu.sync_copy(zv.at[pl.ds(0, nr)], dst_sp.at[pl.ds(r0, nr)])
    if layer1:
        pltpu.sync_copy(ones_h, onesv)

    pltpu.sync_copy(zr_h, rows)
    pltpu.sync_copy(zv_h, zv)
    for k in range(4):
        r0 = nbase + k * 1024
        nr = 1024 if k < 3 else NODE_SLICE - 3 * 1024
        pltpu.sync_copy(rows.at[pl.ds(0, nr), :], out_sp.at[pl.ds(r0, nr), :])
        pltpu.sync_copy(zv.at[pl.ds(0, nr)], den_sp.at[pl.ds(r0, nr)])
        if layer1:
            pltpu.sync_copy(zv.at[pl.ds(0, nr)], cnt_sp.at[pl.ds(r0, nr)])
            pltpu.sync_copy(zv.at[pl.ds(0, nr)], s0_sp.at[pl.ds(r0, nr)])
            pltpu.sync_copy(zv.at[pl.ds(0, nr)], s1_sp.at[pl.ds(r0, nr)])
            pltpu.sync_copy(zv.at[pl.ds(0, nr)], s2_sp.at[pl.ds(r0, nr)])

    plsc.subcore_barrier()

    def make_chunk(compute, coff):
        def chunk(ch, _):
            srcv, srcv2, dstv, cv, av, adv, wv, ev, rows, ea0v, ea1v, ea2v = bufs[0]
            gsem, nsem = gsems[0], nsems[0]
            base = s * ROWS_PER_SUB + ch * CH_ROWS
            pltpu.sync_copy(src2d.at[pl.ds(base, CH_ROWS)], srcv)
            pltpu.sync_copy(dst2d.at[pl.ds(base, CH_ROWS)], dstv)
            if compute:
                pltpu.sync_copy(c2d.at[pl.ds(base, CH_ROWS)], cv)
            else:
                pltpu.sync_copy(e2d.at[pl.ds(base, CH_ROWS)], cv)
            for j in range(CH_ROWS):
                for k in range(8):
                    srcv2[j, pl.ds(k * 16, 16)] = srcv[j, pl.ds(k * 16, 16)] + coff
            gds = [pltpu.async_copy(hs_h.at[srcv2.at[j]],
                                    rows.at[pl.ds(j * 128, 128), :], gsem)
                   for j in range(CH_ROWS)]
            if compute:
                if layer1:
                    pltpu.sync_copy(ea0_h.at[pl.ds(base, CH_ROWS)], ea0v)
                    pltpu.sync_copy(ea1_h.at[pl.ds(base, CH_ROWS)], ea1v)
                    pltpu.sync_copy(ea2_h.at[pl.ds(base, CH_ROWS)], ea2v)
                nds = []
                for j in range(CH_ROWS):
                    nds.append(pltpu.async_copy(asrc_sp.at[srcv.at[j]], av.at[j], nsem))
                    nds.append(pltpu.async_copy(adst_sp.at[dstv.at[j]], adv.at[j], nsem))
                    if not layer1:
                        nds.append(pltpu.async_copy(winv_sp.at[dstv.at[j]], wv.at[j], nsem))
                for d in nds:
                    d.wait()
                for j in range(CH_ROWS):
                    for k in range(8):
                        sl = pl.ds(k * 16, 16)
                        if layer1:
                            al = av[j, sl] + adv[j, sl] + cv[j, sl]
                        else:
                            al = av[j, sl] + adv[j, sl] + cv[j, sl] * wv[j, sl]
                        al = jnp.where(al > 0, al, 0.2 * al)
                        ev[j, sl] = jnp.exp(al)

                @pl.when(c == 0)
                def _():
                    pltpu.sync_copy(ev, e2d.at[pl.ds(base, CH_ROWS)])
                    for j in range(CH_ROWS):
                        pltpu.sync_copy(ev.at[j], den_sp.at[dstv.at[j]], add=True)
                    if layer1:
                        for j in range(CH_ROWS):
                            pltpu.sync_copy(onesv.at[j], cnt_sp.at[dstv.at[j]], add=True)
                            pltpu.sync_copy(ea0v.at[j], s0_sp.at[dstv.at[j]], add=True)
                            pltpu.sync_copy(ea1v.at[j], s1_sp.at[dstv.at[j]], add=True)
                            pltpu.sync_copy(ea2v.at[j], s2_sp.at[dstv.at[j]], add=True)
            else:
                for j in range(CH_ROWS):
                    for k in range(8):
                        sl = pl.ds(k * 16, 16)
                        ev[j, sl] = cv[j, sl]

            for d in gds:
                d.wait()
            for j in range(CH_ROWS):
                for g in range(8):
                    e16 = ev[j, pl.ds(g * 16, 16)]
                    for i in range(16):
                        es = jnp.broadcast_to(e16[i:i + 1], (16,))
                        r = j * 128 + g * 16 + i
                        rows[r, :] = rows[r, :] * es
            for j in range(CH_ROWS):
                pltpu.sync_copy(rows.at[pl.ds(j * 128, 128), :],
                                out_sp.at[dstv.at[j]], add=True)
            return 0

        return chunk

    def epilogue_out(qbase):
        for k in range(4):
            r0 = nbase + k * 1024
            nr = 1024 if k < 3 else NODE_SLICE - 3 * 1024
            pltpu.sync_copy(out_sp.at[pl.ds(r0, nr), :], rows.at[pl.ds(0, nr), :])
            pltpu.sync_copy(rows.at[pl.ds(0, nr), :], out_h.at[pl.ds(qbase + r0, nr), :])

    lax.fori_loop(0, N_CHUNKS, make_chunk(True, c * NPAD), 0)
    plsc.subcore_barrier()
    epilogue_out(c * NPAD)

    @pl.when(c == 0)
    def _():
        for k in range(4):
            r0 = nbase + k * 1024
            nr = 1024 if k < 3 else NODE_SLICE - 3 * 1024
            pltpu.sync_copy(den_sp.at[pl.ds(r0, nr)], zv.at[pl.ds(0, nr)])
            pltpu.sync_copy(zv.at[pl.ds(0, nr)], den_h.at[pl.ds(r0, nr)])
            if layer1:
                pltpu.sync_copy(cnt_sp.at[pl.ds(r0, nr)], zv.at[pl.ds(0, nr)])
                pltpu.sync_copy(zv.at[pl.ds(0, nr)], aux0_h.at[pl.ds(r0, nr)])
                pltpu.sync_copy(s0_sp.at[pl.ds(r0, nr)], zv.at[pl.ds(0, nr)])
                pltpu.sync_copy(zv.at[pl.ds(0, nr)], aux1_h.at[pl.ds(r0, nr)])
                pltpu.sync_copy(s1_sp.at[pl.ds(r0, nr)], zv.at[pl.ds(0, nr)])
                pltpu.sync_copy(zv.at[pl.ds(0, nr)], aux2_h.at[pl.ds(r0, nr)])
                pltpu.sync_copy(s2_sp.at[pl.ds(r0, nr)], zv.at[pl.ds(0, nr)])
                pltpu.sync_copy(zv.at[pl.ds(0, nr)], aux3_h.at[pl.ds(r0, nr)])

    if two_pass:
        pltpu.sync_copy(zr_h, rows)
        for k in range(4):
            r0 = nbase + k * 1024
            nr = 1024 if k < 3 else NODE_SLICE - 3 * 1024
            pltpu.sync_copy(rows.at[pl.ds(0, nr), :], out_sp.at[pl.ds(r0, nr), :])
        plsc.subcore_barrier()
        lax.fori_loop(0, N_CHUNKS, make_chunk(False, (2 + c) * NPAD), 0)
        plsc.subcore_barrier()
        epilogue_out((2 + c) * NPAD)


@functools.partial(jax.jit, static_argnames=("layer1", "two_pass"))
def _sc_sweep(src2d, dst2d, c2d, asrc, adst, winv, hs, ea0, ea1, ea2,
              *, layer1, two_pass):
    mesh = plsc.VectorSubcoreMesh(core_axis_name="c", subcore_axis_name="s")
    if layer1:
        sp_names = ["asrc", "adst", "den", "cnt", "s0", "s1", "s2"]
    else:
        sp_names = ["asrc", "adst", "winv", "den"]
    nq = 4 if two_pass else 2
    body = functools.partial(_sweep_body, layer1, two_pass, sp_names)
    zr = jnp.zeros((1024, WH), jnp.float32)
    zvv = jnp.zeros((1024,), jnp.float32)
    ones2d = jnp.ones((CH_ROWS, 128), jnp.float32)
    f = pl.kernel(
        body,
        out_type=[
            jax.ShapeDtypeStruct((EROWS, 128), jnp.float32),
            jax.ShapeDtypeStruct((NPAD,), jnp.float32),
            jax.ShapeDtypeStruct((nq * NPAD, WH), jnp.float32),
            jax.ShapeDtypeStruct((NPAD,), jnp.float32),
            jax.ShapeDtypeStruct((NPAD,), jnp.float32),
            jax.ShapeDtypeStruct((NPAD,), jnp.float32),
            jax.ShapeDtypeStruct((NPAD,), jnp.float32),
        ],
        mesh=mesh,
        compiler_params=pltpu.CompilerParams(use_tc_tiling_on_sc=False),
        scratch_types=(
            [
                pltpu.VMEM((CH_ROWS, 128), jnp.int32),
                pltpu.VMEM((CH_ROWS, 128), jnp.int32),
                pltpu.VMEM((CH_ROWS, 128), jnp.int32),
                pltpu.VMEM((CH_ROWS, 128), jnp.float32),
                pltpu.VMEM((CH_ROWS, 128), jnp.float32),
                pltpu.VMEM((CH_ROWS, 128), jnp.float32),
                pltpu.VMEM((CH_ROWS, 128), jnp.float32),
                pltpu.VMEM((CH_ROWS, 128), jnp.float32),
                pltpu.VMEM((1024, WH), jnp.float32),
                pltpu.VMEM((CH_ROWS, 128), jnp.float32),
                pltpu.VMEM((CH_ROWS, 128), jnp.float32),
                pltpu.VMEM((CH_ROWS, 128), jnp.float32),
            ] * 2
            + [
                pltpu.VMEM((1024,), jnp.float32),
                pltpu.VMEM((CH_ROWS, 128), jnp.float32),
                pltpu.VMEM_SHARED((NPAD, WH), jnp.float32),
            ]
            + [pltpu.VMEM_SHARED((NPAD,), jnp.float32) for _ in sp_names]
            + [pltpu.SemaphoreType.DMA] * 5
        ),
    )
    return f(src2d, dst2d, c2d, asrc, adst, winv, hs, ea0, ea1, ea2, zr, zvv, ones2d)


def _pad_nodes(v):
    return jnp.pad(v, (0, NPAD - N))


def _prep_h(h):
    hp = jnp.pad(h, ((0, NPAD - N), (0, 0)))
    nq = h.shape[1] // WH
    return jnp.concatenate([hp[:, q * WH:(q + 1) * WH] for q in range(nq)], axis=0)


def _edge2d(v, fill=0):
    return jnp.pad(v, (0, EPAD - E), constant_values=fill).reshape(EROWS, 128)


def _layer(src2d, dst2d, c2d, h, asrc, adst, winv, ea_cols, layer1):
    hs = _prep_h(h)
    nq = h.shape[1] // WH
    e2d, den, out, a0, a1, a2, a3 = _sc_sweep(
        src2d, dst2d, c2d, _pad_nodes(asrc), _pad_nodes(adst), _pad_nodes(winv),
        hs, ea_cols[0], ea_cols[1], ea_cols[2],
        layer1=layer1, two_pass=(nq == 4))
    num = jnp.concatenate([out[q * NPAD:q * NPAD + N, :] for q in range(nq)], axis=1)
    return e2d, den[:N], num, (a0, a1, a2, a3)


def _leaky_exp(a):
    return jnp.exp(jnp.where(a > 0, a, 0.2 * a))


def kernel(x, edge_index, edge_attr, batch,
           conv1_W, conv1_We, conv1_as, conv1_ad, conv1_ae, conv1_b,
           conv2_W, conv2_We, conv2_as, conv2_ad, conv2_ae, conv2_b,
           conv3_W, conv3_We, conv3_as, conv3_ad, conv3_ae, conv3_b):
    n = x.shape[0]
    src2d = _edge2d(edge_index[0], 0)
    dst2d = _edge2d(edge_index[1], N)
    ea_cols = [_edge2d(edge_attr[:, i]) for i in range(3)]
    ones_n = jnp.ones((n,), jnp.float32)

    h1 = x @ conv1_W
    as1 = h1 @ conv1_as
    ad1 = h1 @ conv1_ad
    w1 = conv1_We @ conv1_ae
    c1 = _edge2d(edge_attr @ w1)
    e1, den1e, num1, aux = _layer(src2d, dst2d, c1, h1, as1, ad1, ones_n, ea_cols, True)
    indeg = aux[0][:N]
    loop_attr1 = jnp.stack([aux[1][:N], aux[2][:N], aux[3][:N]], axis=1) / jnp.maximum(indeg, 1.0)[:, None]
    ediag1 = _leaky_exp(as1 + ad1 + loop_attr1 @ w1)
    den1 = den1e + ediag1
    out1 = (num1 + h1 * ediag1[:, None]) / (den1[:, None] + 1e-16) + conv1_b
    hr1 = jax.nn.relu(out1)
    inv1 = 1.0 / (den1 + 1e-16)
    a1_diag = ediag1 * inv1

    h2 = hr1 @ conv2_W
    as2 = h2 @ conv2_as
    ad2 = h2 @ conv2_ad
    w2 = (conv2_We @ conv2_ae)[0]
    loop_attr2 = 1.0 / (indeg + 1.0)
    e2, den2e, num2, _ = _layer(src2d, dst2d, e1, h2, as2, ad2, w2 * inv1, ea_cols, False)
    eA = _leaky_exp(as2 + ad2 + w2 * a1_diag)
    eB = _leaky_exp(as2 + ad2 + w2 * loop_attr2)
    den2 = den2e + eA + eB
    out2 = (num2 + h2 * (eA + eB)[:, None]) / (den2[:, None] + 1e-16) + conv2_b
    hr2 = jax.nn.relu(out2)
    inv2 = 1.0 / (den2 + 1e-16)
    a2_diagA = eA * inv2
    a2_diagB = eB * inv2

    h3 = hr2 @ conv3_W
    as3 = h3 @ conv3_as
    ad3 = h3 @ conv3_ad
    w3 = (conv3_We @ conv3_ae)[0]
    loop_attr3 = 1.0 / (indeg + 2.0)
    _, den3e, num3, _ = _layer(src2d, dst2d, e2, h3, as3, ad3, w3 * inv2, ea_cols, False)
    eC = _leaky_exp(as3 + ad3 + w3 * a2_diagA)
    eD = _leaky_exp(as3 + ad3 + w3 * a2_diagB)
    eE = _leaky_exp(as3 + ad3 + w3 * loop_attr3)
    den3 = den3e + eC + eD + eE
    out3 = (num3 + h3 * (eC + eD + eE)[:, None]) / (den3[:, None] + 1e-16) + conv3_b

    ssum = jax.ops.segment_sum(out3, batch, num_segments=_N_GRAPHS, indices_are_sorted=True)
    cnt = jax.ops.segment_sum(ones_n, batch, num_segments=_N_GRAPHS, indices_are_sorted=True)
    return ssum / jnp.maximum(cnt, 1.0)[:, None]


# speedup vs baseline: 1.0531x; 1.0531x over previous
import functools

import jax
import jax.numpy as jnp
from jax import lax
from jax.experimental import pallas as pl
from jax.experimental.pallas import tpu as pltpu
from jax.experimental.pallas import tpu_sc as plsc

_N_GRAPHS = 8
N = 50000
E = 800000
NPAD = 50048
EPAD = 819200
EROWS = EPAD // 128
NS = 16
ROWS_PER_SUB = EROWS // NS
CH_ROWS = 8
N_CHUNKS = ROWS_PER_SUB // CH_ROWS
NODE_SLICE = NPAD // NS
WH = 16


def _sweep_body(layer1, two_pass, sp_names, *args):
    (src2d, dst2d, c2d, asrc_h, adst_h, winv_h, hs_h,
     ea0_h, ea1_h, ea2_h, zr_h, zv_h, ones_h,
     e2d, den_h, out_h, aux0_h, aux1_h, aux2_h, aux3_h) = args[:20]
    vmem = args[20:46]
    out_sp = args[46]
    sp = dict(zip(sp_names, args[47:47 + len(sp_names)]))
    gsem0, gsem1, nsem0, nsem1, osem = args[47 + len(sp_names):]
    asrc_sp = sp.get("asrc")
    adst_sp = sp.get("adst")
    winv_sp = sp.get("winv")
    den_sp = sp.get("den")
    cnt_sp = sp.get("cnt")
    s0_sp = sp.get("s0")
    s1_sp = sp.get("s1")
    s2_sp = sp.get("s2")
    c = lax.axis_index("c")
    s = lax.axis_index("s")
    bufs = [vmem[0:12], vmem[12:24]]
    zv, onesv = vmem[24:26]
    gsems = [gsem0, gsem1]
    nsems = [nsem0, nsem1]
    rows = bufs[0][8]

    nbase = s * NODE_SLICE
    if True:
        pairs = [(asrc_h, asrc_sp), (adst_h, adst_sp)]
        if not layer1:
            pairs.append((winv_h, winv_sp))
        for src_h, dst_sp in pairs:
            for k in range(4):
                r0 = nbase + k * 1024
                nr = 1024 if k < 3 else NODE_SLICE - 3 * 1024
                pltpu.sync_copy(src_h.at[pl.ds(r0, nr)], zv.at[pl.ds(0, nr)])
                pltpu.sync_copy(zv.at[pl.ds(0, nr)], dst_sp.at[pl.ds(r0, nr)])
    if layer1:
        pltpu.sync_copy(ones_h, onesv)

    pltpu.sync_copy(zr_h, rows)
    pltpu.sync_copy(zv_h, zv)
    for k in range(4):
        r0 = nbase + k * 1024
        nr = 1024 if k < 3 else NODE_SLICE - 3 * 1024
        pltpu.sync_copy(rows.at[pl.ds(0, nr), :], out_sp.at[pl.ds(r0, nr), :])
        pltpu.sync_copy(zv.at[pl.ds(0, nr)], den_sp.at[pl.ds(r0, nr)])
        if layer1:
            pltpu.sync_copy(zv.at[pl.ds(0, nr)], cnt_sp.at[pl.ds(r0, nr)])
            pltpu.sync_copy(zv.at[pl.ds(0, nr)], s0_sp.at[pl.ds(r0, nr)])
            pltpu.sync_copy(zv.at[pl.ds(0, nr)], s1_sp.at[pl.ds(r0, nr)])
            pltpu.sync_copy(zv.at[pl.ds(0, nr)], s2_sp.at[pl.ds(r0, nr)])

    plsc.subcore_barrier()

    def make_chunk(compute, coff):
        def chunk(ch, _):
            srcv, srcv2, dstv, cv, av, adv, wv, ev, rows, ea0v, ea1v, ea2v = bufs[0]
            gsem, nsem = gsems[0], nsems[0]
            base = s * ROWS_PER_SUB + ch * CH_ROWS
            pltpu.sync_copy(src2d.at[pl.ds(base, CH_ROWS)], srcv)
            pltpu.sync_copy(dst2d.at[pl.ds(base, CH_ROWS)], dstv)
            if compute:
                pltpu.sync_copy(c2d.at[pl.ds(base, CH_ROWS)], cv)
            else:
                pltpu.sync_copy(e2d.at[pl.ds(base, CH_ROWS)], cv)
            for j in range(CH_ROWS):
                for k in range(8):
                    srcv2[j, pl.ds(k * 16, 16)] = srcv[j, pl.ds(k * 16, 16)] + coff
            gds = [pltpu.async_copy(hs_h.at[srcv2.at[j]],
                                    rows.at[pl.ds(j * 128, 128), :], gsem)
                   for j in range(CH_ROWS)]
            if compute:
                if layer1:
                    pltpu.sync_copy(ea0_h.at[pl.ds(base, CH_ROWS)], ea0v)
                    pltpu.sync_copy(ea1_h.at[pl.ds(base, CH_ROWS)], ea1v)
                    pltpu.sync_copy(ea2_h.at[pl.ds(base, CH_ROWS)], ea2v)
                nds = []
                for j in range(CH_ROWS):
                    nds.append(pltpu.async_copy(asrc_sp.at[srcv.at[j]], av.at[j], nsem))
                    nds.append(pltpu.async_copy(adst_sp.at[dstv.at[j]], adv.at[j], nsem))
                    if not layer1:
                        nds.append(pltpu.async_copy(winv_sp.at[dstv.at[j]], wv.at[j], nsem))
                for d in nds:
                    d.wait()
                for j in range(CH_ROWS):
                    for k in range(8):
                        sl = pl.ds(k * 16, 16)
                        if layer1:
                            al = av[j, sl] + adv[j, sl] + cv[j, sl]
                        else:
                            al = av[j, sl] + adv[j, sl] + cv[j, sl] * wv[j, sl]
                        al = jnp.where(al > 0, al, 0.2 * al)
                        ev[j, sl] = jnp.exp(al)

                @pl.when(c == 0)
                def _():
                    pltpu.sync_copy(ev, e2d.at[pl.ds(base, CH_ROWS)])
                    for j in range(CH_ROWS):
                        pltpu.sync_copy(ev.at[j], den_sp.at[dstv.at[j]], add=True)
                    if layer1:
                        for j in range(CH_ROWS):
                            pltpu.sync_copy(onesv.at[j], cnt_sp.at[dstv.at[j]], add=True)
                            pltpu.sync_copy(ea0v.at[j], s0_sp.at[dstv.at[j]], add=True)
                            pltpu.sync_copy(ea1v.at[j], s1_sp.at[dstv.at[j]], add=True)
                            pltpu.sync_copy(ea2v.at[j], s2_sp.at[dstv.at[j]], add=True)
            else:
                for j in range(CH_ROWS):
                    for k in range(8):
                        sl = pl.ds(k * 16, 16)
                        ev[j, sl] = cv[j, sl]

            for d in gds:
                d.wait()
            for j in range(CH_ROWS):
                for g in range(8):
                    e16 = ev[j, pl.ds(g * 16, 16)]
                    for i in range(16):
                        es = jnp.broadcast_to(e16[i:i + 1], (16,))
                        r = j * 128 + g * 16 + i
                        rows[r, :] = rows[r, :] * es
            ods = [pltpu.async_copy(rows.at[pl.ds(j * 128, 128), :],
                                    out_sp.at[dstv.at[j]], osem, add=True)
                   for j in range(CH_ROWS)]
            for d in ods:
                d.wait()
            return 0

        return chunk

    def epilogue_out(qbase):
        for k in range(4):
            r0 = nbase + k * 1024
            nr = 1024 if k < 3 else NODE_SLICE - 3 * 1024
            pltpu.sync_copy(out_sp.at[pl.ds(r0, nr), :], rows.at[pl.ds(0, nr), :])
            pltpu.sync_copy(rows.at[pl.ds(0, nr), :], out_h.at[pl.ds(qbase + r0, nr), :])

    lax.fori_loop(0, N_CHUNKS, make_chunk(True, c * NPAD), 0)
    plsc.subcore_barrier()
    epilogue_out(c * NPAD)

    @pl.when(c == 0)
    def _():
        for k in range(4):
            r0 = nbase + k * 1024
            nr = 1024 if k < 3 else NODE_SLICE - 3 * 1024
            pltpu.sync_copy(den_sp.at[pl.ds(r0, nr)], zv.at[pl.ds(0, nr)])
            pltpu.sync_copy(zv.at[pl.ds(0, nr)], den_h.at[pl.ds(r0, nr)])
            if layer1:
                pltpu.sync_copy(cnt_sp.at[pl.ds(r0, nr)], zv.at[pl.ds(0, nr)])
                pltpu.sync_copy(zv.at[pl.ds(0, nr)], aux0_h.at[pl.ds(r0, nr)])
                pltpu.sync_copy(s0_sp.at[pl.ds(r0, nr)], zv.at[pl.ds(0, nr)])
                pltpu.sync_copy(zv.at[pl.ds(0, nr)], aux1_h.at[pl.ds(r0, nr)])
                pltpu.sync_copy(s1_sp.at[pl.ds(r0, nr)], zv.at[pl.ds(0, nr)])
                pltpu.sync_copy(zv.at[pl.ds(0, nr)], aux2_h.at[pl.ds(r0, nr)])
                pltpu.sync_copy(s2_sp.at[pl.ds(r0, nr)], zv.at[pl.ds(0, nr)])
                pltpu.sync_copy(zv.at[pl.ds(0, nr)], aux3_h.at[pl.ds(r0, nr)])

    if two_pass:
        pltpu.sync_copy(zr_h, rows)
        for k in range(4):
            r0 = nbase + k * 1024
            nr = 1024 if k < 3 else NODE_SLICE - 3 * 1024
            pltpu.sync_copy(rows.at[pl.ds(0, nr), :], out_sp.at[pl.ds(r0, nr), :])
        plsc.subcore_barrier()
        lax.fori_loop(0, N_CHUNKS, make_chunk(False, (2 + c) * NPAD), 0)
        plsc.subcore_barrier()
        epilogue_out((2 + c) * NPAD)


@functools.partial(jax.jit, static_argnames=("layer1", "two_pass"))
def _sc_sweep(src2d, dst2d, c2d, asrc, adst, winv, hs, ea0, ea1, ea2,
              *, layer1, two_pass):
    mesh = plsc.VectorSubcoreMesh(core_axis_name="c", subcore_axis_name="s")
    if layer1:
        sp_names = ["asrc", "adst", "den", "cnt", "s0", "s1", "s2"]
    else:
        sp_names = ["asrc", "adst", "winv", "den"]
    nq = 4 if two_pass else 2
    body = functools.partial(_sweep_body, layer1, two_pass, sp_names)
    zr = jnp.zeros((1024, WH), jnp.float32)
    zvv = jnp.zeros((1024,), jnp.float32)
    ones2d = jnp.ones((CH_ROWS, 128), jnp.float32)
    f = pl.kernel(
        body,
        out_type=[
            jax.ShapeDtypeStruct((EROWS, 128), jnp.float32),
            jax.ShapeDtypeStruct((NPAD,), jnp.float32),
            jax.ShapeDtypeStruct((nq * NPAD, WH), jnp.float32),
            jax.ShapeDtypeStruct((NPAD,), jnp.float32),
            jax.ShapeDtypeStruct((NPAD,), jnp.float32),
            jax.ShapeDtypeStruct((NPAD,), jnp.float32),
            jax.ShapeDtypeStruct((NPAD,), jnp.float32),
        ],
        mesh=mesh,
        compiler_params=pltpu.CompilerParams(use_tc_tiling_on_sc=False),
        scratch_types=(
            [
                pltpu.VMEM((CH_ROWS, 128), jnp.int32),
                pltpu.VMEM((CH_ROWS, 128), jnp.int32),
                pltpu.VMEM((CH_ROWS, 128), jnp.int32),
                pltpu.VMEM((CH_ROWS, 128), jnp.float32),
                pltpu.VMEM((CH_ROWS, 128), jnp.float32),
                pltpu.VMEM((CH_ROWS, 128), jnp.float32),
                pltpu.VMEM((CH_ROWS, 128), jnp.float32),
                pltpu.VMEM((CH_ROWS, 128), jnp.float32),
                pltpu.VMEM((1024, WH), jnp.float32),
                pltpu.VMEM((CH_ROWS, 128), jnp.float32),
                pltpu.VMEM((CH_ROWS, 128), jnp.float32),
                pltpu.VMEM((CH_ROWS, 128), jnp.float32),
            ] * 2
            + [
                pltpu.VMEM((1024,), jnp.float32),
                pltpu.VMEM((CH_ROWS, 128), jnp.float32),
                pltpu.VMEM_SHARED((NPAD, WH), jnp.float32),
            ]
            + [pltpu.VMEM_SHARED((NPAD,), jnp.float32) for _ in sp_names]
            + [pltpu.SemaphoreType.DMA] * 5
        ),
    )
    return f(src2d, dst2d, c2d, asrc, adst, winv, hs, ea0, ea1, ea2, zr, zvv, ones2d)


def _pad_nodes(v):
    return jnp.pad(v, (0, NPAD - N))


def _prep_h(h):
    hp = jnp.pad(h, ((0, NPAD - N), (0, 0)))
    nq = h.shape[1] // WH
    return jnp.concatenate([hp[:, q * WH:(q + 1) * WH] for q in range(nq)], axis=0)


def _edge2d(v, fill=0):
    return jnp.pad(v, (0, EPAD - E), constant_values=fill).reshape(EROWS, 128)


def _layer(src2d, dst2d, c2d, h, asrc, adst, winv, ea_cols, layer1):
    hs = _prep_h(h)
    nq = h.shape[1] // WH
    e2d, den, out, a0, a1, a2, a3 = _sc_sweep(
        src2d, dst2d, c2d, _pad_nodes(asrc), _pad_nodes(adst), _pad_nodes(winv),
        hs, ea_cols[0], ea_cols[1], ea_cols[2],
        layer1=layer1, two_pass=(nq == 4))
    num = jnp.concatenate([out[q * NPAD:q * NPAD + N, :] for q in range(nq)], axis=1)
    return e2d, den[:N], num, (a0, a1, a2, a3)


def _leaky_exp(a):
    return jnp.exp(jnp.where(a > 0, a, 0.2 * a))


def kernel(x, edge_index, edge_attr, batch,
           conv1_W, conv1_We, conv1_as, conv1_ad, conv1_ae, conv1_b,
           conv2_W, conv2_We, conv2_as, conv2_ad, conv2_ae, conv2_b,
           conv3_W, conv3_We, conv3_as, conv3_ad, conv3_ae, conv3_b):
    n = x.shape[0]
    src2d = _edge2d(edge_index[0], 0)
    dst2d = _edge2d(edge_index[1], N)
    ea_cols = [_edge2d(edge_attr[:, i]) for i in range(3)]
    ones_n = jnp.ones((n,), jnp.float32)

    h1 = x @ conv1_W
    as1 = h1 @ conv1_as
    ad1 = h1 @ conv1_ad
    w1 = conv1_We @ conv1_ae
    c1 = _edge2d(edge_attr @ w1)
    e1, den1e, num1, aux = _layer(src2d, dst2d, c1, h1, as1, ad1, ones_n, ea_cols, True)
    indeg = aux[0][:N]
    loop_attr1 = jnp.stack([aux[1][:N], aux[2][:N], aux[3][:N]], axis=1) / jnp.maximum(indeg, 1.0)[:, None]
    ediag1 = _leaky_exp(as1 + ad1 + loop_attr1 @ w1)
    den1 = den1e + ediag1
    out1 = (num1 + h1 * ediag1[:, None]) / (den1[:, None] + 1e-16) + conv1_b
    hr1 = jax.nn.relu(out1)
    inv1 = 1.0 / (den1 + 1e-16)
    a1_diag = ediag1 * inv1

    h2 = hr1 @ conv2_W
    as2 = h2 @ conv2_as
    ad2 = h2 @ conv2_ad
    w2 = (conv2_We @ conv2_ae)[0]
    loop_attr2 = 1.0 / (indeg + 1.0)
    e2, den2e, num2, _ = _layer(src2d, dst2d, e1, h2, as2, ad2, w2 * inv1, ea_cols, False)
    eA = _leaky_exp(as2 + ad2 + w2 * a1_diag)
    eB = _leaky_exp(as2 + ad2 + w2 * loop_attr2)
    den2 = den2e + eA + eB
    out2 = (num2 + h2 * (eA + eB)[:, None]) / (den2[:, None] + 1e-16) + conv2_b
    hr2 = jax.nn.relu(out2)
    inv2 = 1.0 / (den2 + 1e-16)
    a2_diagA = eA * inv2
    a2_diagB = eB * inv2

    h3 = hr2 @ conv3_W
    as3 = h3 @ conv3_as
    ad3 = h3 @ conv3_ad
    w3 = (conv3_We @ conv3_ae)[0]
    loop_attr3 = 1.0 / (indeg + 2.0)
    _, den3e, num3, _ = _layer(src2d, dst2d, e2, h3, as3, ad3, w3 * inv2, ea_cols, False)
    eC = _leaky_exp(as3 + ad3 + w3 * a2_diagA)
    eD = _leaky_exp(as3 + ad3 + w3 * a2_diagB)
    eE = _leaky_exp(as3 + ad3 + w3 * loop_attr3)
    den3 = den3e + eC + eD + eE
    out3 = (num3 + h3 * (eC + eD + eE)[:, None]) / (den3[:, None] + 1e-16) + conv3_b

    ssum = jax.ops.segment_sum(out3, batch, num_segments=_N_GRAPHS, indices_are_sorted=True)
    cnt = jax.ops.segment_sum(ones_n, batch, num_segments=_N_GRAPHS, indices_are_sorted=True)
    return ssum / jnp.maximum(cnt, 1.0)[:, None]
